# Initial kernel scaffold; baseline (speedup 1.0000x reference)
import jax
import jax.numpy as jnp
from jax.experimental import pallas as pl


def kernel(x, rows, cols, weights):
    raise NotImplementedError("write your pallas kernel here")


# speedup vs baseline: 494.8921x; 494.8921x over previous
import jax
import jax.numpy as jnp
from jax import lax
from jax.experimental import pallas as pl
from jax.experimental.pallas import tpu as pltpu
from jax.experimental.pallas import tpu_sc as plsc

IN_FEATURES = 4096
FEATURES = 128

NC = 2
NS = 16
LANES = 16
NW = NC * NS

CHUNK = 2048
GROUPS = FEATURES // LANES


def _sc_body(x_hbm, rows_hbm, cols_hbm, w_hbm, out_hbm,
             x_v, r_v, c_v, w_v, acc_v):
    cid = lax.axis_index("c")
    sid = lax.axis_index("s")
    wid = sid * NC + cid

    pltpu.sync_copy(x_hbm, x_v)

    nchunks_total = rows_hbm.shape[0] // CHUNK
    per = -(-nchunks_total // NW)
    start_chunk = wid * per
    n = jnp.minimum(per, nchunks_total - start_chunk)

    zero = jnp.zeros((LANES,), jnp.float32)
    acc0 = (zero,) * GROUPS

    def chunk_body(c, acc):
        base = (start_chunk + c) * CHUNK
        pltpu.sync_copy(rows_hbm.at[pl.ds(base, CHUNK)], r_v)
        pltpu.sync_copy(cols_hbm.at[pl.ds(base, CHUNK)], c_v)
        pltpu.sync_copy(w_hbm.at[pl.ds(base, CHUNK)], w_v)

        def row_body(k, acc):
            accl = list(acc)
            for g in range(GROUPS):
                off = k * FEATURES + g * LANES
                ir = r_v[pl.ds(off, LANES)]
                ic = c_v[pl.ds(off, LANES)]
                xr = plsc.load_gather(x_v, [ir])
                xc = plsc.load_gather(x_v, [ic])
                w = w_v[pl.ds(off, LANES)]
                accl[g] = accl[g] + xr * xc * w
            return tuple(accl)

        return lax.fori_loop(0, CHUNK // FEATURES, row_body, acc)

    acc = lax.fori_loop(0, n, chunk_body, acc0)

    for g in range(GROUPS):
        acc_v[pl.ds(g * LANES, LANES)] = acc[g]
    pltpu.sync_copy(acc_v, out_hbm.at[wid])


def kernel(x, rows, cols, weights):
    wflat = weights.reshape(-1)
    mesh = plsc.VectorSubcoreMesh(core_axis_name="c", subcore_axis_name="s")
    kfn = pl.kernel(
        _sc_body,
        out_type=jax.ShapeDtypeStruct((NW, FEATURES), jnp.float32),
        mesh=mesh,
        compiler_params=pltpu.CompilerParams(needs_layout_passes=False),
        scratch_types=[
            pltpu.VMEM((IN_FEATURES,), jnp.float32),
            pltpu.VMEM((CHUNK,), jnp.int32),
            pltpu.VMEM((CHUNK,), jnp.int32),
            pltpu.VMEM((CHUNK,), jnp.float32),
            pltpu.VMEM((FEATURES,), jnp.float32),
        ],
    )
    partial = kfn(x, rows, cols, wflat)
    return partial.sum(axis=0)


# speedup vs baseline: 1354.6910x; 2.7373x over previous
import jax
import jax.numpy as jnp
from jax import lax
from jax.experimental import pallas as pl
from jax.experimental.pallas import tpu as pltpu
from jax.experimental.pallas import tpu_sc as plsc

IN_FEATURES = 4096
FEATURES = 128

NC = 2
NS = 16
LANES = 16
NW = NC * NS

CHUNK = 1024
GROUPS = FEATURES // LANES


def _sc_body(x_hbm, rows_hbm, cols_hbm, w_hbm, out_hbm,
             x_v, r0_v, r1_v, c0_v, c1_v, w0_v, w1_v, acc_v, sem0, sem1):
    cid = lax.axis_index("c")
    sid = lax.axis_index("s")
    wid = sid * NC + cid

    pltpu.sync_copy(x_hbm, x_v)

    nchunks_total = rows_hbm.shape[0] // CHUNK
    per = -(-nchunks_total // NW)
    start_chunk = wid * per
    n = jnp.minimum(per, nchunks_total - start_chunk)

    sems = (sem0, sem1)
    bufs = ((r0_v, c0_v, w0_v), (r1_v, c1_v, w1_v))

    def start_fetch(b, c):
        base = (start_chunk + c) * CHUNK
        rb, cb, wb = bufs[b]
        pltpu.async_copy(rows_hbm.at[pl.ds(base, CHUNK)], rb, sems[b])
        pltpu.async_copy(cols_hbm.at[pl.ds(base, CHUNK)], cb, sems[b])
        pltpu.async_copy(w_hbm.at[pl.ds(base, CHUNK)], wb, sems[b])

    def wait_fetch(b):
        rb, cb, wb = bufs[b]
        pltpu.make_async_copy(rows_hbm.at[pl.ds(0, CHUNK)], rb, sems[b]).wait()
        pltpu.make_async_copy(cols_hbm.at[pl.ds(0, CHUNK)], cb, sems[b]).wait()
        pltpu.make_async_copy(w_hbm.at[pl.ds(0, CHUNK)], wb, sems[b]).wait()

    def compute(b, acc):
        rb, cb, wb = bufs[b]

        def row_body(k, acc):
            accl = list(acc)
            for g in range(GROUPS):
                off = k * FEATURES + g * LANES
                ir = rb[pl.ds(off, LANES)]
                ic = cb[pl.ds(off, LANES)]
                xr = plsc.load_gather(x_v, [ir])
                xc = plsc.load_gather(x_v, [ic])
                w = wb[pl.ds(off, LANES)]
                accl[g] = accl[g] + xr * xc * w
            return tuple(accl)

        return lax.fori_loop(0, CHUNK // FEATURES, row_body, acc)

    start_fetch(0, 0)
    start_fetch(1, 1)

    zero = jnp.zeros((LANES,), jnp.float32)
    acc0 = (zero,) * GROUPS

    def pair_body(g, acc):
        c0 = 2 * g
        wait_fetch(0)

        @pl.when(c0 + 2 < n)
        def _():
            start_fetch(0, c0 + 2)

        acc = compute(0, acc)
        wait_fetch(1)

        @pl.when(c0 + 3 < n)
        def _():
            start_fetch(1, c0 + 3)

        return compute(1, acc)

    acc = lax.fori_loop(0, n // 2, pair_body, acc0)

    for g in range(GROUPS):
        acc_v[pl.ds(g * LANES, LANES)] = acc[g]
    pltpu.sync_copy(acc_v, out_hbm.at[wid])


def kernel(x, rows, cols, weights):
    wflat = weights.reshape(-1)
    mesh = plsc.VectorSubcoreMesh(core_axis_name="c", subcore_axis_name="s")
    kfn = pl.kernel(
        _sc_body,
        out_type=jax.ShapeDtypeStruct((NW, FEATURES), jnp.float32),
        mesh=mesh,
        compiler_params=pltpu.CompilerParams(needs_layout_passes=False),
        scratch_types=[
            pltpu.VMEM((IN_FEATURES,), jnp.float32),
            pltpu.VMEM((CHUNK,), jnp.int32),
            pltpu.VMEM((CHUNK,), jnp.int32),
            pltpu.VMEM((CHUNK,), jnp.int32),
            pltpu.VMEM((CHUNK,), jnp.int32),
            pltpu.VMEM((CHUNK,), jnp.float32),
            pltpu.VMEM((CHUNK,), jnp.float32),
            pltpu.VMEM((FEATURES,), jnp.float32),
            pltpu.SemaphoreType.DMA,
            pltpu.SemaphoreType.DMA,
        ],
    )
    partial = kfn(x, rows, cols, wflat)
    return partial.sum(axis=0)
